# baseline (speedup 1.0000x reference)
import functools

import jax
import jax.numpy as jnp
from jax import lax
from jax.experimental import pallas as pl
from jax.experimental.pallas import tpu as pltpu
from jax.experimental.pallas import tpu_sc as plsc

N_SEG = 2048
L = 16


@functools.lru_cache(maxsize=None)
def _build_sc_kernel(B, F, H, W, CPW):
    RPC = 8
    CHUNK = RPC * W
    NCHUNK = H // RPC
    CPU_ = CPW // 2
    GROUPS = CHUNK // L
    WPB = F // CPW
    assert B * WPB == 32, "mapping assumes 32 SC subcores"

    mesh = plsc.VectorSubcoreMesh(core_axis_name="c", subcore_axis_name="s")

    @functools.partial(
        pl.kernel,
        mesh=mesh,
        compiler_params=pltpu.CompilerParams(needs_layout_passes=False),
        out_type=[
            jax.ShapeDtypeStruct((B * F * N_SEG,), jnp.float32),
            jax.ShapeDtypeStruct((B * N_SEG,), jnp.float32),
            jax.ShapeDtypeStruct((B * 2 * N_SEG,), jnp.float32),
        ],
        scratch_types=[
            pltpu.VMEM((2, RPC, W), jnp.int32),
            pltpu.VMEM((2, CPU_, RPC, W), jnp.float32),
            pltpu.VMEM((2, RPC, W), jnp.float32),
            pltpu.VMEM((CPW * N_SEG,), jnp.float32),
            pltpu.VMEM((N_SEG,), jnp.float32),
            pltpu.VMEM((N_SEG,), jnp.float32),
            pltpu.SemaphoreType.DMA,
            pltpu.SemaphoreType.DMA,
            pltpu.SemaphoreType.DMA,
            pltpu.SemaphoreType.DMA,
        ],
    )
    def sp_kernel(feat_hbm, seg_hbm, coord_hbm, sum_hbm, cnt_hbm, cent_hbm,
                  seg_v, feat_v, coord_v, acc_v, cnt_v, cent_v,
                  fsem0, fsem1, ssem0, ssem1):
        cid = lax.axis_index("c")
        sid = lax.axis_index("s")
        batch = cid * (B // 2) + sid // WPB
        wib = sid % WPB
        c0 = wib * CPW
        is_cent = jnp.logical_or(wib == 1, wib == 2)

        iota = lax.iota(jnp.int32, L)
        zeros = jnp.zeros((L,), jnp.float32)
        ones = jnp.ones((L,), jnp.float32)
        col_base = [jnp.full((L,), cc * N_SEG, jnp.int32) for cc in range(CPW)]

        def zero_small(i, _):
            cnt_v[pl.ds(i * L, L)] = zeros
            cent_v[pl.ds(i * L, L)] = zeros
            return 0
        lax.fori_loop(0, N_SEG // L, zero_small, 0)

        @plsc.parallel_loop(0, (CPW * N_SEG) // L, unroll=8)
        def zero_acc(i):
            acc_v[pl.ds(i * L, L)] = zeros

        fsems = [fsem0, fsem1]
        ssems = [ssem0, ssem1]

        def feat_copies(g, half):
            r0 = g * RPC
            return [pltpu.make_async_copy(
                feat_hbm.at[batch, c0 + half * CPU_ + cc, pl.ds(r0, RPC), :],
                feat_v.at[half, cc], fsems[half])
                for cc in range(CPU_)]

        def seg_copies(g, sl):
            r0 = g * RPC
            return [pltpu.make_async_copy(
                seg_hbm.at[batch, pl.ds(r0, RPC), :], seg_v.at[sl],
                ssems[sl])]

        def coord_copies(g, sl):
            r0 = g * RPC
            return [pltpu.make_async_copy(
                coord_hbm.at[wib - 1, pl.ds(r0, RPC), :], coord_v.at[sl],
                ssems[sl])]

        def start(cps):
            for cp in cps:
                cp.start()

        def wait(cps):
            for cp in cps:
                cp.wait()

        start(seg_copies(0, 0))

        @pl.when(is_cent)
        def _():
            start(coord_copies(0, 0))
        start(feat_copies(0, 0))
        start(feat_copies(0, 1))

        def process_unit(g, half, sl):
            @plsc.parallel_loop(0, GROUPS, unroll=2)
            def grp(i):
                r = i >> 5
                cq = (i & 31) * L
                seg16 = seg_v[sl, r, pl.ds(cq, L)]
                for cc in range(CPU_):
                    val = feat_v[half, cc, r, pl.ds(cq, L)]
                    plsc.addupdate_scatter(
                        acc_v, [seg16 + col_base[half * CPU_ + cc]], val)

        def process_extras(g, sl):
            @pl.when(wib == 0)
            def _():
                @plsc.parallel_loop(0, GROUPS, unroll=4)
                def grpn(i):
                    r = i >> 5
                    cq = (i & 31) * L
                    seg16 = seg_v[sl, r, pl.ds(cq, L)]
                    plsc.addupdate_scatter(cnt_v, [seg16], ones)

            @pl.when(is_cent)
            def _():
                @plsc.parallel_loop(0, GROUPS, unroll=4)
                def grpc(i):
                    r = i >> 5
                    cq = (i & 31) * L
                    seg16 = seg_v[sl, r, pl.ds(cq, L)]
                    v = coord_v[sl, r, pl.ds(cq, L)]
                    plsc.addupdate_scatter(cent_v, [seg16], v)

        def outer(gg, _):
            for j in range(2):
                g = gg * 2 + j
                for half in range(2):
                    wait(feat_copies(g, half))
                    if half == 0:
                        wait(seg_copies(g, j))

                        @pl.when(jnp.logical_and(is_cent, g > 0))
                        def _():
                            wait(coord_copies(g, j))

                        @pl.when(g + 1 < NCHUNK)
                        def _():
                            start(seg_copies(g + 1, 1 - j))

                            @pl.when(is_cent)
                            def _():
                                start(coord_copies(g + 1, 1 - j))
                    process_unit(g, half, j)

                    @pl.when(g + 1 < NCHUNK)
                    def _():
                        start(feat_copies(g + 1, half))
                process_extras(g, j)
            return 0
        lax.fori_loop(0, NCHUNK // 2, outer, 0)

        pltpu.sync_copy(
            acc_v, sum_hbm.at[pl.ds((batch * F + c0) * N_SEG, CPW * N_SEG)])

        @pl.when(wib == 0)
        def _():
            pltpu.sync_copy(cnt_v, cnt_hbm.at[pl.ds(batch * N_SEG, N_SEG)])

        @pl.when(is_cent)
        def _():
            pltpu.sync_copy(
                cent_v,
                cent_hbm.at[pl.ds((batch * 2 + (wib - 1)) * N_SEG, N_SEG)])

    return sp_kernel


@functools.lru_cache(maxsize=None)
def _build_tc_finalize(B, F):
    def fin(sums_ref, cnt_ref, xy_ref, tok_ref, mask_ref, cent_ref):
        cnt = cnt_ref[...]
        recip = 1.0 / jnp.maximum(cnt, 1.0)
        means = sums_ref[...] * recip[None, :]
        tok_ref[0] = means.T
        mask_ref[0, 0] = cnt > 0
        xy = xy_ref[...].reshape(2, N_SEG) * recip[None, :]
        cent_ref[0] = xy.T

    return pl.pallas_call(
        fin,
        grid=(B,),
        in_specs=[
            pl.BlockSpec((F, N_SEG), lambda b: (b, 0)),
            pl.BlockSpec((N_SEG,), lambda b: (b,)),
            pl.BlockSpec((2 * N_SEG,), lambda b: (b,)),
        ],
        out_specs=[
            pl.BlockSpec((1, N_SEG, F), lambda b: (b, 0, 0)),
            pl.BlockSpec((1, 1, N_SEG), lambda b: (b, 0, 0)),
            pl.BlockSpec((1, N_SEG, 2), lambda b: (b, 0, 0)),
        ],
        out_shape=[
            jax.ShapeDtypeStruct((B, N_SEG, F), jnp.float32),
            jax.ShapeDtypeStruct((B, 1, N_SEG), jnp.bool_),
            jax.ShapeDtypeStruct((B, N_SEG, 2), jnp.float32),
        ],
    )


def kernel(images, features, segment_map):
    B, F, H, W = features.shape
    x = jnp.arange(W, dtype=jnp.float32) / (W - 1)
    y = jnp.arange(H, dtype=jnp.float32) / (H - 1)
    coords = jnp.stack([
        jnp.broadcast_to(x[None, :], (H, W)),
        jnp.broadcast_to(y[:, None], (H, W)),
    ])
    sum1d, cnt1d, xy1d = _build_sc_kernel(B, F, H, W, 12)(
        features, segment_map, coords)
    tokens, mask3d, centroids = _build_tc_finalize(B, F)(
        sum1d.reshape(B * F, N_SEG), cnt1d, xy1d)
    return (tokens, segment_map, mask3d.reshape(B, N_SEG), centroids)


# speedup vs baseline: 1.0205x; 1.0205x over previous
import functools

import jax
import jax.numpy as jnp
from jax import lax
from jax.experimental import pallas as pl
from jax.experimental.pallas import tpu as pltpu
from jax.experimental.pallas import tpu_sc as plsc

N_SEG = 2048
L = 16


@functools.lru_cache(maxsize=None)
def _build_sc_kernel(B, F, H, W, CPW):
    RPC = 8
    CHUNK = RPC * W
    NCHUNK = H // RPC
    CPU_ = CPW // 2
    GROUPS = CHUNK // L
    WPB = F // CPW
    assert B * WPB == 32, "mapping assumes 32 SC subcores"

    mesh = plsc.VectorSubcoreMesh(core_axis_name="c", subcore_axis_name="s")

    @functools.partial(
        pl.kernel,
        mesh=mesh,
        compiler_params=pltpu.CompilerParams(needs_layout_passes=False),
        out_type=[
            jax.ShapeDtypeStruct((B * F * N_SEG,), jnp.float32),
            jax.ShapeDtypeStruct((B * N_SEG,), jnp.float32),
            jax.ShapeDtypeStruct((B * 2 * N_SEG,), jnp.float32),
        ],
        scratch_types=[
            pltpu.VMEM((2, RPC, W), jnp.int32),
            pltpu.VMEM((2, CPU_, RPC, W), jnp.float32),
            pltpu.VMEM((2, RPC, W), jnp.float32),
            pltpu.VMEM((CPW * N_SEG,), jnp.float32),
            pltpu.VMEM((N_SEG,), jnp.float32),
            pltpu.VMEM((N_SEG,), jnp.float32),
            pltpu.SemaphoreType.DMA,
            pltpu.SemaphoreType.DMA,
            pltpu.SemaphoreType.DMA,
            pltpu.SemaphoreType.DMA,
        ],
    )
    def sp_kernel(feat_hbm, seg_hbm, coord_hbm, sum_hbm, cnt_hbm, cent_hbm,
                  seg_v, feat_v, coord_v, acc_v, cnt_v, cent_v,
                  fsem0, fsem1, ssem0, ssem1):
        cid = lax.axis_index("c")
        sid = lax.axis_index("s")
        batch = cid * (B // 2) + sid // WPB
        wib = sid % WPB
        c0 = wib * CPW
        is_cent = jnp.logical_or(wib == 1, wib == 2)

        iota = lax.iota(jnp.int32, L)
        zeros = jnp.zeros((L,), jnp.float32)
        ones = jnp.ones((L,), jnp.float32)
        col_base = [jnp.full((L,), cc * N_SEG, jnp.int32) for cc in range(CPW)]

        def zero_small(i, _):
            cnt_v[pl.ds(i * L, L)] = zeros
            cent_v[pl.ds(i * L, L)] = zeros
            return 0
        lax.fori_loop(0, N_SEG // L, zero_small, 0)

        @plsc.parallel_loop(0, (CPW * N_SEG) // L, unroll=8)
        def zero_acc(i):
            acc_v[pl.ds(i * L, L)] = zeros

        fsems = [fsem0, fsem1]
        ssems = [ssem0, ssem1]

        def feat_copies(g, half):
            r0 = g * RPC
            return [pltpu.make_async_copy(
                feat_hbm.at[batch, c0 + half * CPU_ + cc, pl.ds(r0, RPC), :],
                feat_v.at[half, cc], fsems[half])
                for cc in range(CPU_)]

        def seg_copies(g, sl):
            r0 = g * RPC
            return [pltpu.make_async_copy(
                seg_hbm.at[batch, pl.ds(r0, RPC), :], seg_v.at[sl],
                ssems[sl])]

        def coord_copies(g, sl):
            r0 = g * RPC
            return [pltpu.make_async_copy(
                coord_hbm.at[wib - 1, pl.ds(r0, RPC), :], coord_v.at[sl],
                ssems[sl])]

        def start(cps):
            for cp in cps:
                cp.start()

        def wait(cps):
            for cp in cps:
                cp.wait()

        start(seg_copies(0, 0))

        @pl.when(is_cent)
        def _():
            start(coord_copies(0, 0))
        start(feat_copies(0, 0))
        start(feat_copies(0, 1))

        def process_unit(g, half, sl):
            @plsc.parallel_loop(0, GROUPS, unroll=2)
            def grp(i):
                r = i >> 5
                cq = (i & 31) * L
                seg16 = seg_v[sl, r, pl.ds(cq, L)]
                for cc in range(CPU_):
                    val = feat_v[half, cc, r, pl.ds(cq, L)]
                    plsc.addupdate_scatter(
                        acc_v, [seg16 + col_base[half * CPU_ + cc]], val)

        def process_extras(g, sl):
            @pl.when(wib == 0)
            def _():
                @plsc.parallel_loop(0, GROUPS, unroll=4)
                def grpn(i):
                    r = i >> 5
                    cq = (i & 31) * L
                    seg16 = seg_v[sl, r, pl.ds(cq, L)]
                    plsc.addupdate_scatter(cnt_v, [seg16], ones)

            @pl.when(is_cent)
            def _():
                @plsc.parallel_loop(0, GROUPS, unroll=4)
                def grpc(i):
                    r = i >> 5
                    cq = (i & 31) * L
                    seg16 = seg_v[sl, r, pl.ds(cq, L)]
                    v = coord_v[sl, r, pl.ds(cq, L)]
                    plsc.addupdate_scatter(cent_v, [seg16], v)

        def outer(gg, _):
            for j in range(2):
                g = gg * 2 + j
                for half in range(2):
                    wait(feat_copies(g, half))
                    if half == 0:
                        wait(seg_copies(g, j))

                        @pl.when(jnp.logical_and(is_cent, g > 0))
                        def _():
                            wait(coord_copies(g, j))

                        @pl.when(g + 1 < NCHUNK)
                        def _():
                            start(seg_copies(g + 1, 1 - j))

                            @pl.when(is_cent)
                            def _():
                                start(coord_copies(g + 1, 1 - j))
                    process_unit(g, half, j)

                    @pl.when(g + 1 < NCHUNK)
                    def _():
                        start(feat_copies(g + 1, half))
                process_extras(g, j)
            return 0
        lax.fori_loop(0, NCHUNK // 2, outer, 0)

        pltpu.sync_copy(
            acc_v, sum_hbm.at[pl.ds((batch * F + c0) * N_SEG, CPW * N_SEG)])

        @pl.when(wib == 0)
        def _():
            pltpu.sync_copy(cnt_v, cnt_hbm.at[pl.ds(batch * N_SEG, N_SEG)])

        @pl.when(is_cent)
        def _():
            pltpu.sync_copy(
                cent_v,
                cent_hbm.at[pl.ds((batch * 2 + (wib - 1)) * N_SEG, N_SEG)])

    return sp_kernel


@functools.lru_cache(maxsize=None)
def _build_tc_finalize(B, F):
    def fin(sums_ref, cnt_ref, xy_ref, tok_ref, mask_ref, cent_ref):
        cnt = cnt_ref[...]
        recip = 1.0 / jnp.maximum(cnt, 1.0)
        tok_ref[0] = sums_ref[...] * recip[None, :]
        mask_ref[0, 0] = cnt > 0
        cent_ref[0] = xy_ref[...].reshape(2, N_SEG) * recip[None, :]

    return pl.pallas_call(
        fin,
        grid=(B,),
        in_specs=[
            pl.BlockSpec((F, N_SEG), lambda b: (b, 0)),
            pl.BlockSpec((N_SEG,), lambda b: (b,)),
            pl.BlockSpec((2 * N_SEG,), lambda b: (b,)),
        ],
        out_specs=[
            pl.BlockSpec((1, F, N_SEG), lambda b: (b, 0, 0)),
            pl.BlockSpec((1, 1, N_SEG), lambda b: (b, 0, 0)),
            pl.BlockSpec((1, 2, N_SEG), lambda b: (b, 0, 0)),
        ],
        out_shape=[
            jax.ShapeDtypeStruct((B, F, N_SEG), jnp.float32),
            jax.ShapeDtypeStruct((B, 1, N_SEG), jnp.bool_),
            jax.ShapeDtypeStruct((B, 2, N_SEG), jnp.float32),
        ],
    )


def kernel(images, features, segment_map):
    B, F, H, W = features.shape
    x = jnp.arange(W, dtype=jnp.float32) / (W - 1)
    y = jnp.arange(H, dtype=jnp.float32) / (H - 1)
    coords = jnp.stack([
        jnp.broadcast_to(x[None, :], (H, W)),
        jnp.broadcast_to(y[:, None], (H, W)),
    ])
    sum1d, cnt1d, xy1d = _build_sc_kernel(B, F, H, W, 12)(
        features, segment_map, coords)
    tok_cm, mask3d, cent_cm = _build_tc_finalize(B, F)(
        sum1d.reshape(B * F, N_SEG), cnt1d, xy1d)
    return (tok_cm.transpose(0, 2, 1), segment_map,
            mask3d.reshape(B, N_SEG), cent_cm.transpose(0, 2, 1))


# speedup vs baseline: 1.0731x; 1.0515x over previous
import functools

import jax
import jax.numpy as jnp
from jax import lax
from jax.experimental import pallas as pl
from jax.experimental.pallas import tpu as pltpu
from jax.experimental.pallas import tpu_sc as plsc

N_SEG = 2048
L = 16


@functools.lru_cache(maxsize=None)
def _build_sc_kernel(B, F, H, W, CPW):
    RPC = 8
    CHUNK = RPC * W
    NCHUNK = H // RPC
    CPU_ = CPW // 2
    GROUPS = CHUNK // L
    WPB = F // CPW
    assert B * WPB == 32, "mapping assumes 32 SC subcores"

    mesh = plsc.VectorSubcoreMesh(core_axis_name="c", subcore_axis_name="s")

    @functools.partial(
        pl.kernel,
        mesh=mesh,
        compiler_params=pltpu.CompilerParams(needs_layout_passes=False),
        out_type=[
            jax.ShapeDtypeStruct((B * F * N_SEG,), jnp.float32),
            jax.ShapeDtypeStruct((B * 8 * N_SEG,), jnp.float32),
            jax.ShapeDtypeStruct((B * 8 * 2 * N_SEG,), jnp.float32),
        ],
        scratch_types=[
            pltpu.VMEM((2, RPC, W), jnp.int32),
            pltpu.VMEM((2, CPU_, RPC, W), jnp.float32),
            pltpu.VMEM((2, RPC, W), jnp.float32),
            pltpu.VMEM((CPW * N_SEG,), jnp.float32),
            pltpu.VMEM((N_SEG,), jnp.float32),
            pltpu.VMEM((N_SEG,), jnp.float32),
            pltpu.VMEM((N_SEG,), jnp.float32),
            pltpu.SemaphoreType.DMA,
            pltpu.SemaphoreType.DMA,
            pltpu.SemaphoreType.DMA,
            pltpu.SemaphoreType.DMA,
        ],
    )
    def sp_kernel(feat_hbm, seg_hbm, coord_hbm, sum_hbm, cnt_hbm, cent_hbm,
                  seg_v, feat_v, coord_v, acc_v, cnt_v, centx_v, centy_v,
                  fsem0, fsem1, ssem0, ssem1):
        cid = lax.axis_index("c")
        sid = lax.axis_index("s")
        batch = cid * (B // 2) + sid // WPB
        wib = sid % WPB
        c0 = wib * CPW
        def coord_cond(g):
            return jnp.logical_or((g + 1) % WPB == wib, (g + 2) % WPB == wib)

        def coord_plane(g):
            return jnp.where((g + 1) % WPB == wib, 0, 1)

        iota = lax.iota(jnp.int32, L)
        zeros = jnp.zeros((L,), jnp.float32)
        ones = jnp.ones((L,), jnp.float32)
        col_base = [jnp.full((L,), cc * N_SEG, jnp.int32) for cc in range(CPW)]

        def zero_small(i, _):
            cnt_v[pl.ds(i * L, L)] = zeros
            centx_v[pl.ds(i * L, L)] = zeros
            centy_v[pl.ds(i * L, L)] = zeros
            return 0
        lax.fori_loop(0, N_SEG // L, zero_small, 0)

        @plsc.parallel_loop(0, (CPW * N_SEG) // L, unroll=8)
        def zero_acc(i):
            acc_v[pl.ds(i * L, L)] = zeros

        fsems = [fsem0, fsem1]
        ssems = [ssem0, ssem1]

        def feat_copies(g, half):
            r0 = g * RPC
            return [pltpu.make_async_copy(
                feat_hbm.at[batch, c0 + half * CPU_ + cc, pl.ds(r0, RPC), :],
                feat_v.at[half, cc], fsems[half])
                for cc in range(CPU_)]

        def seg_copies(g, sl):
            r0 = g * RPC
            return [pltpu.make_async_copy(
                seg_hbm.at[batch, pl.ds(r0, RPC), :], seg_v.at[sl],
                ssems[sl])]

        def coord_copies(g, sl):
            r0 = g * RPC
            return [pltpu.make_async_copy(
                coord_hbm.at[coord_plane(g), pl.ds(r0, RPC), :],
                coord_v.at[sl], ssems[sl])]

        def start(cps):
            for cp in cps:
                cp.start()

        def wait(cps):
            for cp in cps:
                cp.wait()

        start(seg_copies(0, 0))

        @pl.when(coord_cond(0))
        def _():
            start(coord_copies(0, 0))
        start(feat_copies(0, 0))
        start(feat_copies(0, 1))

        def process_unit(g, half, sl):
            @plsc.parallel_loop(0, GROUPS, unroll=2)
            def grp(i):
                r = i >> 5
                cq = (i & 31) * L
                seg16 = seg_v[sl, r, pl.ds(cq, L)]
                for cc in range(CPU_):
                    val = feat_v[half, cc, r, pl.ds(cq, L)]
                    plsc.addupdate_scatter(
                        acc_v, [seg16 + col_base[half * CPU_ + cc]], val)

        def process_extras(g, sl):
            @pl.when(g % WPB == wib)
            def _():
                @plsc.parallel_loop(0, GROUPS, unroll=4)
                def grpn(i):
                    r = i >> 5
                    cq = (i & 31) * L
                    seg16 = seg_v[sl, r, pl.ds(cq, L)]
                    plsc.addupdate_scatter(cnt_v, [seg16], ones)

            @pl.when((g + 1) % WPB == wib)
            def _():
                @plsc.parallel_loop(0, GROUPS, unroll=4)
                def grpx(i):
                    r = i >> 5
                    cq = (i & 31) * L
                    seg16 = seg_v[sl, r, pl.ds(cq, L)]
                    v = coord_v[sl, r, pl.ds(cq, L)]
                    plsc.addupdate_scatter(centx_v, [seg16], v)

            @pl.when((g + 2) % WPB == wib)
            def _():
                @plsc.parallel_loop(0, GROUPS, unroll=4)
                def grpy(i):
                    r = i >> 5
                    cq = (i & 31) * L
                    seg16 = seg_v[sl, r, pl.ds(cq, L)]
                    v = coord_v[sl, r, pl.ds(cq, L)]
                    plsc.addupdate_scatter(centy_v, [seg16], v)

        def outer(gg, _):
            for j in range(2):
                g = gg * 2 + j
                for half in range(2):
                    wait(feat_copies(g, half))
                    if half == 0:
                        wait(seg_copies(g, j))

                        @pl.when(jnp.logical_and(coord_cond(g), g > 0))
                        def _():
                            wait(coord_copies(g, j))

                        @pl.when(g + 1 < NCHUNK)
                        def _():
                            start(seg_copies(g + 1, 1 - j))

                            @pl.when(coord_cond(g + 1))
                            def _():
                                start(coord_copies(g + 1, 1 - j))
                    process_unit(g, half, j)

                    @pl.when(g + 1 < NCHUNK)
                    def _():
                        start(feat_copies(g + 1, half))
                process_extras(g, j)
            return 0
        lax.fori_loop(0, NCHUNK // 2, outer, 0)

        pltpu.sync_copy(
            acc_v, sum_hbm.at[pl.ds((batch * F + c0) * N_SEG, CPW * N_SEG)])

        wslot = batch * WPB + wib
        pltpu.sync_copy(cnt_v, cnt_hbm.at[pl.ds(wslot * N_SEG, N_SEG)])
        pltpu.sync_copy(centx_v, cent_hbm.at[pl.ds(wslot * 2 * N_SEG, N_SEG)])
        pltpu.sync_copy(
            centy_v, cent_hbm.at[pl.ds((wslot * 2 + 1) * N_SEG, N_SEG)])

    return sp_kernel


@functools.lru_cache(maxsize=None)
def _build_tc_finalize(B, F):
    def fin(sums_ref, cnt_ref, xy_ref, tok_ref, mask_ref, cent_ref):
        cnt = jnp.sum(cnt_ref[0], axis=0)
        recip = 1.0 / jnp.maximum(cnt, 1.0)
        tok_ref[0] = sums_ref[...] * recip[None, :]
        mask_ref[0, 0] = cnt > 0
        xy = jnp.sum(xy_ref[0].reshape(8, 2, N_SEG), axis=0)
        cent_ref[0] = xy * recip[None, :]

    return pl.pallas_call(
        fin,
        grid=(B,),
        in_specs=[
            pl.BlockSpec((F, N_SEG), lambda b: (b, 0)),
            pl.BlockSpec((1, 8, N_SEG), lambda b: (b, 0, 0)),
            pl.BlockSpec((1, 16, N_SEG), lambda b: (b, 0, 0)),
        ],
        out_specs=[
            pl.BlockSpec((1, F, N_SEG), lambda b: (b, 0, 0)),
            pl.BlockSpec((1, 1, N_SEG), lambda b: (b, 0, 0)),
            pl.BlockSpec((1, 2, N_SEG), lambda b: (b, 0, 0)),
        ],
        out_shape=[
            jax.ShapeDtypeStruct((B, F, N_SEG), jnp.float32),
            jax.ShapeDtypeStruct((B, 1, N_SEG), jnp.bool_),
            jax.ShapeDtypeStruct((B, 2, N_SEG), jnp.float32),
        ],
    )


def kernel(images, features, segment_map):
    B, F, H, W = features.shape
    x = jnp.arange(W, dtype=jnp.float32) / (W - 1)
    y = jnp.arange(H, dtype=jnp.float32) / (H - 1)
    coords = jnp.stack([
        jnp.broadcast_to(x[None, :], (H, W)),
        jnp.broadcast_to(y[:, None], (H, W)),
    ])
    sum1d, cnt1d, xy1d = _build_sc_kernel(B, F, H, W, 12)(
        features, segment_map, coords)
    tok_cm, mask3d, cent_cm = _build_tc_finalize(B, F)(
        sum1d.reshape(B * F, N_SEG), cnt1d.reshape(B, 8, N_SEG),
        xy1d.reshape(B, 16, N_SEG))
    return (tok_cm.transpose(0, 2, 1), segment_map,
            mask3d.reshape(B, N_SEG), cent_cm.transpose(0, 2, 1))
